# baseline (speedup 1.0000x reference)
import jax
import jax.numpy as jnp
from jax.experimental import pallas as pl

_B, _D, _H, _R = 16384, 2048, 128, 16
_BT = 512


def _router_body(x_ref, w1_ref, b1_ref, w2_ref, b2_ref, sel_ref, prob_ref):
    h = jnp.dot(x_ref[...], w1_ref[...], preferred_element_type=jnp.float32)
    h = jnp.maximum(h + b1_ref[...], 0.0)
    logits = jnp.dot(h, w2_ref[...], preferred_element_type=jnp.float32)
    logits = logits + b2_ref[...]
    m = jnp.max(logits, axis=-1, keepdims=True)
    e = jnp.exp(logits - m)
    prob_ref[...] = e / jnp.sum(e, axis=-1, keepdims=True)
    iota = jax.lax.broadcasted_iota(jnp.int32, logits.shape, 1)
    sel = jnp.min(jnp.where(logits == m, iota, _R), axis=-1)
    sel_ref[...] = sel[:, None]


def kernel(x, W1, b1, W2, b2, route_bias):
    b1r = b1.reshape(1, _H)
    b2r = (b2 + route_bias).reshape(1, _R)
    grid = (_B // _BT,)
    sel2d, probs = pl.pallas_call(
        _router_body,
        grid=grid,
        in_specs=[
            pl.BlockSpec((_BT, _D), lambda i: (i, 0)),
            pl.BlockSpec((_D, _H), lambda i: (0, 0)),
            pl.BlockSpec((1, _H), lambda i: (0, 0)),
            pl.BlockSpec((_H, _R), lambda i: (0, 0)),
            pl.BlockSpec((1, _R), lambda i: (0, 0)),
        ],
        out_specs=[
            pl.BlockSpec((_BT, 1), lambda i: (i, 0)),
            pl.BlockSpec((_BT, _R), lambda i: (i, 0)),
        ],
        out_shape=[
            jax.ShapeDtypeStruct((_B, 1), jnp.int32),
            jax.ShapeDtypeStruct((_B, _R), jnp.float32),
        ],
    )(x, W1, b1r, W2, b2r)
    return (sel2d.reshape(_B), probs)


# speedup vs baseline: 1.1987x; 1.1987x over previous
import jax
import jax.numpy as jnp
from jax.experimental import pallas as pl

_B, _D, _H, _R = 16384, 2048, 128, 16
_BT = 2048


def _router_body(x_ref, w1_ref, b1_ref, w2_ref, b2_ref, sel_ref, prob_ref):
    h = jnp.dot(x_ref[...], w1_ref[...], preferred_element_type=jnp.float32)
    h = jnp.maximum(h + b1_ref[...], 0.0)
    logits = jnp.dot(h, w2_ref[...], preferred_element_type=jnp.float32)
    logits = logits + b2_ref[...]
    m = jnp.max(logits, axis=-1, keepdims=True)
    e = jnp.exp(logits - m)
    prob_ref[...] = e / jnp.sum(e, axis=-1, keepdims=True)
    iota = jax.lax.broadcasted_iota(jnp.int32, logits.shape, 1)
    sel = jnp.min(jnp.where(logits == m, iota, _R), axis=-1)
    sel_ref[...] = sel[:, None]


def kernel(x, W1, b1, W2, b2, route_bias):
    b1r = b1.reshape(1, _H)
    b2r = (b2 + route_bias).reshape(1, _R)
    grid = (_B // _BT,)
    sel2d, probs = pl.pallas_call(
        _router_body,
        grid=grid,
        in_specs=[
            pl.BlockSpec((_BT, _D), lambda i: (i, 0)),
            pl.BlockSpec((_D, _H), lambda i: (0, 0)),
            pl.BlockSpec((1, _H), lambda i: (0, 0)),
            pl.BlockSpec((_H, _R), lambda i: (0, 0)),
            pl.BlockSpec((1, _R), lambda i: (0, 0)),
        ],
        out_specs=[
            pl.BlockSpec((_BT, 1), lambda i: (i, 0)),
            pl.BlockSpec((_BT, _R), lambda i: (i, 0)),
        ],
        out_shape=[
            jax.ShapeDtypeStruct((_B, 1), jnp.int32),
            jax.ShapeDtypeStruct((_B, _R), jnp.float32),
        ],
    )(x, W1, b1r, W2, b2r)
    return (sel2d.reshape(_B), probs)
